# baseline (speedup 1.0000x reference)
import functools

import jax
import jax.numpy as jnp
from jax import lax
from jax.experimental import pallas as pl
from jax.experimental.pallas import tpu as pltpu
from jax.experimental.pallas import tpu_sc as plsc

MIN_RATING = 1.0
MAX_RATING = 5.0

B = 16384
D = 32
NC = 2
NS = 16
NW = NC * NS
BPW = B // NW
CHUNK = 128
NCH = BPW // CHUNK
L = 16
NG = BPW // L

_mesh = plsc.VectorSubcoreMesh(core_axis_name="c", subcore_axis_name="s")


@functools.partial(
    pl.kernel,
    out_type=jax.ShapeDtypeStruct((B,), jnp.float32),
    mesh=_mesh,
    scratch_types=[
        pltpu.VMEM((NCH, CHUNK), jnp.int32),
        pltpu.VMEM((NCH, CHUNK), jnp.int32),
        pltpu.VMEM((BPW, D), jnp.float32),
        pltpu.VMEM((BPW, D), jnp.float32),
        pltpu.VMEM((BPW, 1), jnp.float32),
        pltpu.VMEM((BPW, 1), jnp.float32),
        pltpu.VMEM((BPW,), jnp.float32),
        pltpu.SemaphoreType.DMA,
    ],
    compiler_params=pltpu.CompilerParams(
        needs_layout_passes=False, use_tc_tiling_on_sc=False),
)
def _lmf_sc(uidx_hbm, midx_hbm, uw_hbm, ub_hbm, mw_hbm, mb_hbm, out_hbm,
            uidx_v, midx_v, uw_v, mw_v, ub_v, mb_v, out_v, sem):
    wid = lax.axis_index("s") * NC + lax.axis_index("c")
    base = wid * BPW

    pltpu.sync_copy(uidx_hbm.at[pl.ds(wid * NCH, NCH)], uidx_v)
    pltpu.sync_copy(midx_hbm.at[pl.ds(wid * NCH, NCH)], midx_v)

    copies = []
    for j in range(NCH):
        sl = pl.ds(j * CHUNK, CHUNK)
        copies.append(pltpu.async_copy(uw_hbm.at[uidx_v.at[j]], uw_v.at[sl], sem))
        copies.append(pltpu.async_copy(mw_hbm.at[midx_v.at[j]], mw_v.at[sl], sem))
        copies.append(pltpu.async_copy(ub_hbm.at[uidx_v.at[j]], ub_v.at[sl], sem))
        copies.append(pltpu.async_copy(mb_hbm.at[midx_v.at[j]], mb_v.at[sl], sem))
    for c in copies:
        c.wait()

    def group(g, carry):
        rows = lax.iota(jnp.int32, L) + g * L
        zero = jnp.zeros((L,), jnp.int32)
        acc = plsc.load_gather(ub_v, [rows, zero]) + plsc.load_gather(mb_v, [rows, zero])
        for j in range(D):
            col = jnp.full((L,), j, jnp.int32)
            u = plsc.load_gather(uw_v, [rows, col])
            m = plsc.load_gather(mw_v, [rows, col])
            acc = acc + u * m
        sl = pl.ds(g * L, L)
        x = acc
        y = 1.0 / (1.0 + jnp.exp(-x))
        out_v[sl] = y * (MAX_RATING - MIN_RATING) + MIN_RATING
        return carry

    lax.fori_loop(0, NG, group, None)
    pltpu.sync_copy(out_v, out_hbm.at[pl.ds(base, BPW)])


def kernel(users, movies, user_weights, user_bias, movie_weights, movie_bias):
    uidx = users.reshape(-1).astype(jnp.int32).reshape(NW * NCH, CHUNK)
    midx = movies.reshape(-1).astype(jnp.int32).reshape(NW * NCH, CHUNK)
    out = _lmf_sc(uidx, midx,
                  user_weights, user_bias,
                  movie_weights, movie_bias)
    return out.reshape(B, 1)


# speedup vs baseline: 2.5774x; 2.5774x over previous
import functools

import jax
import jax.numpy as jnp
from jax import lax
from jax.experimental import pallas as pl
from jax.experimental.pallas import tpu as pltpu
from jax.experimental.pallas import tpu_sc as plsc

MIN_RATING = 1.0
MAX_RATING = 5.0

B = 16384
D = 32
NC = 2
NS = 16
NW = NC * NS
BPW = B // NW
HALF = BPW // 2
CHUNK = 128
NCH = HALF // CHUNK
L = 16
NG = HALF // L

_mesh = plsc.VectorSubcoreMesh(core_axis_name="c", subcore_axis_name="s")


@functools.partial(
    pl.kernel,
    out_type=jax.ShapeDtypeStruct((B,), jnp.float32),
    mesh=_mesh,
    scratch_types=[
        pltpu.VMEM((4, CHUNK), jnp.int32),
        pltpu.VMEM((4, CHUNK), jnp.int32),
        pltpu.VMEM((4, CHUNK), jnp.int32),
        pltpu.VMEM((4, CHUNK), jnp.int32),
        pltpu.VMEM((HALF, CHUNK), jnp.float32),
        pltpu.VMEM((HALF, CHUNK), jnp.float32),
        pltpu.VMEM((BPW,), jnp.float32),
        pltpu.VMEM((BPW,), jnp.float32),
        pltpu.VMEM((BPW,), jnp.float32),
        pltpu.SemaphoreType.DMA,
    ],
    compiler_params=pltpu.CompilerParams(
        needs_layout_passes=False, use_tc_tiling_on_sc=False),
)
def _lmf_sc(uidx_hbm, midx_hbm, uw_hbm, ub_hbm, mw_hbm, mb_hbm, out_hbm,
            uidx_v, midx_v, urow_v, mrow_v, uw_v, mw_v, ub_v, mb_v, out_v,
            sem):
    wid = lax.axis_index("s") * NC + lax.axis_index("c")
    base = wid * BPW

    pltpu.sync_copy(uidx_hbm.at[pl.ds(wid * 4, 4)], uidx_v)
    pltpu.sync_copy(midx_hbm.at[pl.ds(wid * 4, 4)], midx_v)

    def shift_rows(q, carry):
        for r in range(4):
            sl = pl.ds(q * L, L)
            urow_v[r, sl] = jax.lax.shift_right_logical(uidx_v[r, sl], 2)
            mrow_v[r, sl] = jax.lax.shift_right_logical(midx_v[r, sl], 2)
        return carry
    lax.fori_loop(0, CHUNK // L, shift_rows, None)

    ub1 = ub_hbm.at[0]
    mb1 = mb_hbm.at[0]
    bias_copies = []
    for j in range(4):
        sl = pl.ds(j * CHUNK, CHUNK)
        bias_copies.append(pltpu.async_copy(ub1.at[uidx_v.at[j]], ub_v.at[sl], sem))
        bias_copies.append(pltpu.async_copy(mb1.at[midx_v.at[j]], mb_v.at[sl], sem))

    def half(h, w_copies):
        new_copies = []
        for j in range(NCH):
            q = h * NCH + j
            sl = pl.ds(j * CHUNK, CHUNK)
            new_copies.append(pltpu.async_copy(uw_hbm.at[urow_v.at[q]], uw_v.at[sl], sem))
            new_copies.append(pltpu.async_copy(mw_hbm.at[mrow_v.at[q]], mw_v.at[sl], sem))
        for c in new_copies:
            c.wait()

        def group(g, carry):
            rows = lax.iota(jnp.int32, L) + g * L
            sl = pl.ds(h * HALF + g * L, L)
            q = (h * HALF + g * L) // CHUNK
            qsl = pl.ds((h * HALF + g * L) % CHUNK, L)
            ucol0 = (uidx_v[q, qsl] & 3) * D
            mcol0 = (midx_v[q, qsl] & 3) * D
            acc = ub_v[sl] + mb_v[sl]
            for j in range(D):
                u = plsc.load_gather(uw_v, [rows, ucol0 + j])
                m = plsc.load_gather(mw_v, [rows, mcol0 + j])
                acc = acc + u * m
            y = 1.0 / (1.0 + jnp.exp(-acc))
            out_v[sl] = y * (MAX_RATING - MIN_RATING) + MIN_RATING
            return carry

        lax.fori_loop(0, NG, group, None)
        return w_copies

    for c in bias_copies:
        c.wait()
    for h in range(2):
        half(h, None)

    pltpu.sync_copy(out_v, out_hbm.at[pl.ds(base, BPW)])


def kernel(users, movies, user_weights, user_bias, movie_weights, movie_bias):
    uidx = users.reshape(-1).astype(jnp.int32).reshape(NW * 4, CHUNK)
    midx = movies.reshape(-1).astype(jnp.int32).reshape(NW * 4, CHUNK)
    uw4 = user_weights.reshape(-1).reshape(user_weights.shape[0] // 4, 4 * D)
    mw4 = movie_weights.reshape(-1).reshape(movie_weights.shape[0] // 4, 4 * D)
    out = _lmf_sc(uidx, midx, uw4, user_bias.T, mw4, movie_bias.T)
    return out.reshape(B, 1)
